# Initial kernel scaffold; baseline (speedup 1.0000x reference)
import jax
import jax.numpy as jnp
from jax.experimental import pallas as pl


def kernel(x, dw, db, pw, pb):
    raise NotImplementedError("write your pallas kernel here")


# speedup vs baseline: 2.4495x; 2.4495x over previous
import functools

import jax
import jax.numpy as jnp
from jax.experimental import pallas as pl
from jax.experimental.pallas import tpu as pltpu


def _shifted(x, off, length):
    if off == 0:
        return x
    b, c, _ = x.shape
    if off < 0:
        zc = jnp.zeros((b, c, -off), jnp.float32)
        return jnp.concatenate([zc, x[:, :, : length + off]], axis=2)
    zc = jnp.zeros((b, c, off), jnp.float32)
    return jnp.concatenate([x[:, :, off:], zc], axis=2)


def _conv_stats_kernel(x_ref, dw_ref, pw_ref, z_ref, st_ref, *, ksize, b_tile):
    x = x_ref[...]
    dw = dw_ref[...]
    pw = pw_ref[...]
    _, c_in, length = x.shape
    pad = (ksize - 1) // 2

    y = None
    for k in range(ksize):
        tap = _shifted(x, k - pad, length) * dw[:, k].reshape(1, c_in, 1)
        y = tap if y is None else y + tap

    c_out = pw.shape[0]
    s1 = jnp.zeros((c_out, 1), jnp.float32)
    s2 = jnp.zeros((c_out, 1), jnp.float32)
    for b in range(b_tile):
        z = jnp.dot(pw, y[b], preferred_element_type=jnp.float32)
        s1 = s1 + jnp.sum(z, axis=1, keepdims=True)
        s2 = s2 + jnp.sum(z * z, axis=1, keepdims=True)
        z_ref[b] = z.astype(jnp.bfloat16)
    st_ref[:, 0:1] = s1
    st_ref[:, 1:2] = s2


def _apply_kernel(z_ref, sc_ref, sh_ref, o_ref):
    z = z_ref[...].astype(jnp.float32)
    sc = sc_ref[...].reshape(1, -1, 1)
    sh = sh_ref[...].reshape(1, -1, 1)
    o_ref[...] = jnp.maximum(z * sc + sh, 0.0)


@functools.partial(jax.jit, static_argnames=("ksize", "eps"))
def _fused(x, dw, pw, *, ksize, eps):
    n, c_in, length = x.shape
    c_out = pw.shape[0]
    l_out = length

    b1 = 8
    nb1 = n // b1
    b2 = 16
    nb2 = n // b2

    cparams = pltpu.CompilerParams(
        dimension_semantics=("parallel",),
        vmem_limit_bytes=48 * 1024 * 1024,
    )

    kfn1 = functools.partial(_conv_stats_kernel, ksize=ksize, b_tile=b1)
    z, stats = pl.pallas_call(
        kfn1,
        grid=(nb1,),
        out_shape=[
            jax.ShapeDtypeStruct((n, c_out, l_out), jnp.bfloat16),
            jax.ShapeDtypeStruct((nb1, c_out, 2), jnp.float32),
        ],
        in_specs=[
            pl.BlockSpec((b1, c_in, length), lambda blk: (blk, 0, 0)),
            pl.BlockSpec((c_in, ksize), lambda blk: (0, 0)),
            pl.BlockSpec((c_out, c_in), lambda blk: (0, 0)),
        ],
        out_specs=[
            pl.BlockSpec((b1, c_out, l_out), lambda blk: (blk, 0, 0)),
            pl.BlockSpec((None, c_out, 2), lambda blk: (blk, 0, 0)),
        ],
        compiler_params=cparams,
        cost_estimate=pl.CostEstimate(
            flops=n * l_out * (2 * c_in * ksize + 2 * c_out * c_in + 4 * c_out),
            transcendentals=0,
            bytes_accessed=4 * n * c_in * length + 2 * n * c_out * l_out,
        ),
    )(x, dw, pw)

    s = jnp.sum(stats, axis=0)
    count = float(n * l_out)
    mean = s[:, 0] / count
    var = jnp.maximum(s[:, 1] / count - mean * mean, 0.0)
    inv_std = jax.lax.rsqrt(var + eps)
    scale = inv_std.reshape(c_out, 1)
    shift = (-mean * inv_std).reshape(c_out, 1)

    out = pl.pallas_call(
        _apply_kernel,
        grid=(nb2,),
        out_shape=jax.ShapeDtypeStruct((n, c_out, l_out), jnp.float32),
        in_specs=[
            pl.BlockSpec((b2, c_out, l_out), lambda blk: (blk, 0, 0)),
            pl.BlockSpec((c_out, 1), lambda blk: (0, 0)),
            pl.BlockSpec((c_out, 1), lambda blk: (0, 0)),
        ],
        out_specs=pl.BlockSpec((b2, c_out, l_out), lambda blk: (blk, 0, 0)),
        compiler_params=cparams,
        cost_estimate=pl.CostEstimate(
            flops=3 * n * c_out * l_out,
            transcendentals=0,
            bytes_accessed=2 * n * c_out * l_out + 4 * n * c_out * l_out,
        ),
    )(z, scale, shift)
    return out


def kernel(x, dw, db, pw, pb):
    del db, pb
    n, c_in, length = x.shape
    ksize = dw.reshape(c_in, -1).shape[-1]
    c_out = pw.shape[0]
    x = x.astype(jnp.float32)
    dw = dw.astype(jnp.float32).reshape(c_in, ksize)
    pw = pw.astype(jnp.float32).reshape(c_out, c_in)
    return _fused(x, dw, pw, ksize=ksize, eps=1e-5)


# speedup vs baseline: 2.5699x; 1.0492x over previous
import functools

import jax
import jax.numpy as jnp
from jax.experimental import pallas as pl
from jax.experimental.pallas import tpu as pltpu


def _shifted(x, off, length):
    if off == 0:
        return x
    b, c, _ = x.shape
    if off < 0:
        zc = jnp.zeros((b, c, -off), jnp.float32)
        return jnp.concatenate([zc, x[:, :, : length + off]], axis=2)
    zc = jnp.zeros((b, c, off), jnp.float32)
    return jnp.concatenate([x[:, :, off:], zc], axis=2)


def _phased_kernel(x_ref, dw_ref, pw_ref, o_ref, z_scr, st_scr, *,
                   ksize, b_tile, count, eps):
    p = pl.program_id(0)
    b = pl.program_id(1)

    @pl.when(p == 0)
    def _conv():
        x = x_ref[...]
        dw = dw_ref[...]
        pw = pw_ref[...]
        _, c_in, length = x.shape
        pad = (ksize - 1) // 2
        y = None
        for k in range(ksize):
            tap = _shifted(x, k - pad, length) * dw[:, k].reshape(1, c_in, 1)
            y = tap if y is None else y + tap
        for i in range(b_tile):
            z = jnp.dot(pw, y[i], preferred_element_type=jnp.float32)
            z_scr[pl.ds(b * b_tile + i, 1)] = z.astype(jnp.bfloat16)[None]

    @pl.when(p == 1)
    def _stats():
        @pl.when(b == 0)
        def _init():
            st_scr[...] = jnp.zeros_like(st_scr)
        z = z_scr[pl.ds(b * b_tile, b_tile)].astype(jnp.float32)
        s1 = jnp.sum(jnp.sum(z, axis=2, keepdims=True), axis=0)
        s2 = jnp.sum(jnp.sum(z * z, axis=2, keepdims=True), axis=0)
        st_scr[:, 0:1] = st_scr[:, 0:1] + s1
        st_scr[:, 1:2] = st_scr[:, 1:2] + s2

    @pl.when(p == 2)
    def _apply():
        mean = st_scr[:, 0:1] * (1.0 / count)
        var = jnp.maximum(st_scr[:, 1:2] * (1.0 / count) - mean * mean, 0.0)
        inv = jax.lax.rsqrt(var + eps)
        c_out = inv.shape[0]
        scale = inv.reshape(1, c_out, 1)
        shift = (-mean * inv).reshape(1, c_out, 1)
        z = z_scr[pl.ds(b * b_tile, b_tile)].astype(jnp.float32)
        o_ref[...] = jnp.maximum(z * scale + shift, 0.0)


@functools.partial(jax.jit, static_argnames=("ksize", "eps"))
def _fused(x, dw, pw, *, ksize, eps):
    n, c_in, length = x.shape
    c_out = pw.shape[0]
    l_out = length

    b_tile = 8
    nb = n // b_tile

    kfn = functools.partial(_phased_kernel, ksize=ksize, b_tile=b_tile,
                            count=float(n * l_out), eps=eps)
    out = pl.pallas_call(
        kfn,
        grid=(3, nb),
        out_shape=jax.ShapeDtypeStruct((n, c_out, l_out), jnp.float32),
        in_specs=[
            pl.BlockSpec((b_tile, c_in, length),
                         lambda p, b: (b * jnp.where(p == 0, 1, 0), 0, 0)),
            pl.BlockSpec((c_in, ksize), lambda p, b: (0, 0)),
            pl.BlockSpec((c_out, c_in), lambda p, b: (0, 0)),
        ],
        out_specs=pl.BlockSpec((b_tile, c_out, l_out),
                               lambda p, b: (b * jnp.where(p == 2, 1, 0), 0, 0)),
        scratch_shapes=[
            pltpu.VMEM((n, c_out, l_out), jnp.bfloat16),
            pltpu.VMEM((c_out, 8), jnp.float32),
        ],
        compiler_params=pltpu.CompilerParams(
            dimension_semantics=("arbitrary", "arbitrary"),
            vmem_limit_bytes=56 * 1024 * 1024,
        ),
        cost_estimate=pl.CostEstimate(
            flops=n * l_out * (2 * c_in * ksize + 2 * c_out * c_in + 7 * c_out),
            transcendentals=0,
            bytes_accessed=4 * n * c_in * length + 4 * n * c_out * l_out,
        ),
    )(x, dw, pw)
    return out


def kernel(x, dw, db, pw, pb):
    del db, pb
    n, c_in, length = x.shape
    ksize = dw.reshape(c_in, -1).shape[-1]
    c_out = pw.shape[0]
    x = x.astype(jnp.float32)
    dw = dw.astype(jnp.float32).reshape(c_in, ksize)
    pw = pw.astype(jnp.float32).reshape(c_out, c_in)
    return _fused(x, dw, pw, ksize=ksize, eps=1e-5)
